# baseline (speedup 1.0000x reference)
import functools

import jax
import jax.numpy as jnp
from jax import lax
from jax.experimental import pallas as pl
from jax.experimental.pallas import tpu as pltpu
from jax.experimental.pallas import tpu_sc as plsc

N_CELLS = 512
N_GENES = 1000
D = 128

_info = plsc.get_sparse_core_info()
_NC, _NS, _L = _info.num_cores, _info.num_subcores, _info.num_lanes
_NW = _NC * _NS
_BPW = 40
_NACT = N_GENES // _BPW


def _gather_sc(weight1, bias1, idx):
    mesh = plsc.VectorSubcoreMesh(core_axis_name="c", subcore_axis_name="s")

    @functools.partial(
        pl.kernel,
        mesh=mesh,
        out_type=(jax.ShapeDtypeStruct((N_GENES, D), jnp.float32),
                  jax.ShapeDtypeStruct((N_GENES,), jnp.float32)),
        scratch_types=[
            pltpu.VMEM((_BPW,), jnp.int32),
            pltpu.VMEM((_BPW, D), jnp.float32),
            pltpu.VMEM((_BPW,), jnp.float32),
            pltpu.SemaphoreType.DMA,
        ],
    )
    def k(w_hbm, b_hbm, idx_hbm, wout_hbm, bout_hbm,
          idx_v, rows_v, bsel_v, sem):
        wid = lax.axis_index("s") * _NC + lax.axis_index("c")
        base = wid * _BPW

        @pl.when(wid < _NACT)
        def _():
            pltpu.sync_copy(idx_hbm.at[pl.ds(base, _BPW)], idx_v)
            pltpu.async_copy(w_hbm.at[idx_v], rows_v, sem).wait()
            pltpu.sync_copy(rows_v, wout_hbm.at[pl.ds(base, _BPW)])
            pltpu.async_copy(b_hbm.at[idx_v], bsel_v, sem).wait()
            pltpu.sync_copy(bsel_v, bout_hbm.at[pl.ds(base, _BPW)])

    return k(weight1, bias1, idx)


_CB = 32
_CA = 64


def _tca_body(gixr_ref, gixc_ref, w1_ref, b1_ref, e_ref, out_ref, wsc, bsc):
    @pl.when(pl.program_id(0) == 0)
    def _():
        gr = jnp.broadcast_to(gixr_ref[...], (N_GENES, N_GENES))
        gc = jnp.broadcast_to(gixc_ref[...], (N_GENES, N_GENES))
        it0 = lax.broadcasted_iota(jnp.int32, (N_GENES, N_GENES), 0)
        it1 = lax.broadcasted_iota(jnp.int32, (N_GENES, N_GENES), 1)
        m1 = jnp.where(gr == it0, 1.0, 0.0)
        mt = jnp.where(gc == it1, 1.0, 0.0)
        wsc[...] = jnp.dot(mt, w1_ref[...], preferred_element_type=jnp.float32)
        bsc[...] = jnp.dot(b1_ref[...], m1, preferred_element_type=jnp.float32)

    prod = e_ref[...] * wsc[...][None, :, :]
    out_ref[...] = jnp.sum(jnp.swapaxes(prod, 1, 2), axis=1) + bsc[...]


def _tc_body(w_ref, b_ref, e_ref, out_ref):
    prod = e_ref[...] * w_ref[...][None, :, :]
    out_ref[...] = jnp.sum(jnp.swapaxes(prod, 1, 2), axis=1) + b_ref[...]


def kernel(cell_gene_embedding, gene_ix, weight1, bias1):
    w_gath, b_gath = _gather_sc(weight1, bias1, gene_ix)
    b2 = b_gath.reshape(1, N_GENES)

    gixr = gene_ix.reshape(1, N_GENES)
    gixc = gene_ix.reshape(N_GENES, 1)
    b1r = bias1.reshape(1, N_GENES)
    out_a = pl.pallas_call(
        _tca_body,
        grid=(_CA // _CB,),
        in_specs=[
            pl.BlockSpec((1, N_GENES), lambda i: (0, 0)),
            pl.BlockSpec((N_GENES, 1), lambda i: (0, 0)),
            pl.BlockSpec((N_GENES, D), lambda i: (0, 0)),
            pl.BlockSpec((1, N_GENES), lambda i: (0, 0)),
            pl.BlockSpec((_CB, N_GENES, D), lambda i: (i, 0, 0)),
        ],
        out_specs=pl.BlockSpec((_CB, N_GENES), lambda i: (i, 0)),
        out_shape=jax.ShapeDtypeStruct((_CA, N_GENES), jnp.float32),
        scratch_shapes=[
            pltpu.VMEM((N_GENES, D), jnp.float32),
            pltpu.VMEM((1, N_GENES), jnp.float32),
        ],
    )(gixr, gixc, weight1, b1r, cell_gene_embedding)

    out_b = pl.pallas_call(
        _tc_body,
        grid=((N_CELLS - _CA) // _CB,),
        in_specs=[
            pl.BlockSpec((N_GENES, D), lambda i: (0, 0)),
            pl.BlockSpec((1, N_GENES), lambda i: (0, 0)),
            pl.BlockSpec((_CB, N_GENES, D),
                         lambda i: (i + _CA // _CB, 0, 0)),
        ],
        out_specs=pl.BlockSpec((_CB, N_GENES), lambda i: (i, 0)),
        out_shape=jax.ShapeDtypeStruct((N_CELLS - _CA, N_GENES), jnp.float32),
    )(w_gath, b2, cell_gene_embedding)
    return jnp.concatenate([out_a, out_b], axis=0)


# speedup vs baseline: 1.0549x; 1.0549x over previous
import functools

import jax
import jax.numpy as jnp
from jax import lax
from jax.experimental import pallas as pl
from jax.experimental.pallas import tpu as pltpu
from jax.experimental.pallas import tpu_sc as plsc

N_CELLS = 512
N_GENES = 1000
D = 128

_info = plsc.get_sparse_core_info()
_NC, _NS, _L = _info.num_cores, _info.num_subcores, _info.num_lanes
_NW = _NC * _NS
_BPW = 40
_NACT = N_GENES // _BPW


def _gather_sc(weight1, bias1, idx):
    mesh = plsc.VectorSubcoreMesh(core_axis_name="c", subcore_axis_name="s")

    @functools.partial(
        pl.kernel,
        mesh=mesh,
        out_type=(jax.ShapeDtypeStruct((N_GENES, D), jnp.float32),
                  jax.ShapeDtypeStruct((N_GENES,), jnp.float32)),
        scratch_types=[
            pltpu.VMEM((_BPW,), jnp.int32),
            pltpu.VMEM((_BPW, D), jnp.float32),
            pltpu.VMEM((_BPW,), jnp.float32),
            pltpu.SemaphoreType.DMA,
            pltpu.SemaphoreType.DMA,
            pltpu.SemaphoreType.DMA,
        ],
    )
    def k(w_hbm, b_hbm, idx_hbm, wout_hbm, bout_hbm,
          idx_v, rows_v, bsel_v, semw, semb, semo):
        wid = lax.axis_index("s") * _NC + lax.axis_index("c")
        base = wid * _BPW

        @pl.when(wid < _NACT)
        def _():
            pltpu.sync_copy(idx_hbm.at[pl.ds(base, _BPW)], idx_v)
            cw = pltpu.async_copy(w_hbm.at[idx_v], rows_v, semw)
            cb = pltpu.async_copy(b_hbm.at[idx_v], bsel_v, semb)
            cw.wait()
            ow = pltpu.async_copy(rows_v, wout_hbm.at[pl.ds(base, _BPW)], semo)
            cb.wait()
            pltpu.sync_copy(bsel_v, bout_hbm.at[pl.ds(base, _BPW)])
            ow.wait()

    return k(weight1, bias1, idx)


_CB = 32


def _tc_body(w_ref, b_ref, e_ref, out_ref):
    prod = e_ref[...] * w_ref[...][None, :, :]
    out_ref[...] = jnp.sum(jnp.swapaxes(prod, 1, 2), axis=1) + b_ref[...]


def kernel(cell_gene_embedding, gene_ix, weight1, bias1):
    w_gath, b_gath = _gather_sc(weight1, bias1, gene_ix)
    b2 = b_gath.reshape(1, N_GENES)

    out = pl.pallas_call(
        _tc_body,
        grid=(N_CELLS // _CB,),
        in_specs=[
            pl.BlockSpec((N_GENES, D), lambda i: (0, 0)),
            pl.BlockSpec((1, N_GENES), lambda i: (0, 0)),
            pl.BlockSpec((_CB, N_GENES, D), lambda i: (i, 0, 0)),
        ],
        out_specs=pl.BlockSpec((_CB, N_GENES), lambda i: (i, 0)),
        out_shape=jax.ShapeDtypeStruct((N_CELLS, N_GENES), jnp.float32),
    )(w_gath, b2, cell_gene_embedding)
    return out
